# baseline (speedup 1.0000x reference)
import functools

import jax
import jax.numpy as jnp
from jax import lax
from jax.experimental import pallas as pl
from jax.experimental.pallas import tpu as pltpu
from jax.experimental.pallas import tpu_sc as plsc

NUM_SEG = 128
DIM = 512
LANES = 16
VECS = DIM // LANES
NUM_ROWS = 100000
CHUNK = 32
NUM_CHUNKS = NUM_ROWS // CHUNK
NUM_CORES = 2
NUM_SUBCORES = 16
NUM_WORKERS = NUM_CORES * NUM_SUBCORES
ITERS = -(-NUM_CHUNKS // NUM_WORKERS)

_mesh = plsc.VectorSubcoreMesh(core_axis_name="c", subcore_axis_name="s")


@functools.partial(
    pl.kernel,
    mesh=_mesh,
    out_type=(
        jax.ShapeDtypeStruct((NUM_WORKERS, NUM_SEG, DIM), jnp.float32),
        jax.ShapeDtypeStruct((NUM_WORKERS, NUM_SEG, LANES), jnp.float32),
    ),
    scratch_types=[
        pltpu.VMEM((CHUNK, DIM), jnp.float32),
        pltpu.VMEM((CHUNK, DIM), jnp.float32),
        pltpu.VMEM((CHUNK,), jnp.int32),
        pltpu.VMEM((CHUNK,), jnp.int32),
        pltpu.VMEM((NUM_SEG, DIM), jnp.float32),
        pltpu.VMEM((NUM_SEG, LANES), jnp.float32),
        pltpu.SemaphoreType.DMA,
        pltpu.SemaphoreType.DMA,
        pltpu.SemaphoreType.DMA,
        pltpu.SemaphoreType.DMA,
    ],
)
def _segment_sums(x_hbm, b_hbm, sums_hbm, cnts_hbm,
                  buf0, buf1, bidx0, bidx1, acc, cnt,
                  sx0, sx1, sb0, sb1):
    cid = lax.axis_index("c")
    sid = lax.axis_index("s")
    wid = sid * NUM_CORES + cid

    ones16 = jnp.full((LANES,), 1.0, jnp.float32)
    zero16 = jnp.zeros((LANES,), jnp.float32)

    def _zero_seg(g, carry):
        for j in range(VECS):
            acc[g, pl.ds(j * LANES, LANES)] = zero16
        cnt[g] = zero16
        return carry

    lax.fori_loop(0, NUM_SEG, _zero_seg, 0)

    def _start(k, buf, bidx, sx, sb):
        @pl.when(k < 0)
        def _():
            r0 = k * CHUNK
            pltpu.async_copy(b_hbm.at[pl.ds(r0, CHUNK)], bidx, sb)
            pltpu.async_copy(x_hbm.at[pl.ds(r0, CHUNK), :], buf, sx)

    def _finish(k, buf, bidx, sx, sb):
        @pl.when(k < NUM_CHUNKS)
        def _():

            def _group(g, c2):
                bvec = bidx[pl.ds(g * LANES, LANES)]
                b_first = bvec[0]
                b_last = bvec[LANES - 1]

                @pl.when(b_first == b_last)
                def _():
                    for jb in range(0, VECS, 8):
                        regs = []
                        for j in range(jb, jb + 8):
                            regs.append(buf[g * LANES, pl.ds(j * LANES, LANES)])
                        for r in range(1, LANES):
                            row = g * LANES + r
                            for j in range(jb, jb + 8):
                                v = buf[row, pl.ds(j * LANES, LANES)]
                                regs[j - jb] = regs[j - jb] + v
                        for j in range(jb, jb + 8):
                            plsc.addupdate(
                                acc.at[b_first, pl.ds(j * LANES, LANES)],
                                regs[j - jb])
                    plsc.addupdate(cnt.at[b_first], ones16 * float(LANES))

                @pl.when(b_first != b_last)
                def _():
                    for r in range(LANES):
                        b = bvec[r]
                        row = g * LANES + r
                        for j in range(VECS):
                            v = buf[row, pl.ds(j * LANES, LANES)]
                            plsc.addupdate(
                                acc.at[b, pl.ds(j * LANES, LANES)], v)
                        plsc.addupdate(cnt.at[b], ones16)

                return c2

            lax.fori_loop(0, CHUNK // LANES, _group, 0)

    _start(wid, buf0, bidx0, sx0, sb0)

    def _outer(t, carry):
        k0 = wid + NUM_WORKERS * (2 * t)
        k1 = wid + NUM_WORKERS * (2 * t + 1)
        _start(k1, buf1, bidx1, sx1, sb1)
        _finish(k0, buf0, bidx0, sx0, sb0)
        _start(k0 + 2 * NUM_WORKERS, buf0, bidx0, sx0, sb0)
        _finish(k1, buf1, bidx1, sx1, sb1)
        return carry

    lax.fori_loop(0, ITERS // 2, _outer, 0)

    pltpu.sync_copy(acc, sums_hbm.at[wid])
    pltpu.sync_copy(cnt, cnts_hbm.at[wid])


def _combine_body(s_ref, c_ref, o_ref):
    s = jnp.sum(s_ref[...], axis=0)
    c = jnp.sum(c_ref[...], axis=0)[:, 0:1]
    o_ref[...] = s / jnp.maximum(c, 1.0)


_combine = pl.pallas_call(
    _combine_body,
    out_shape=jax.ShapeDtypeStruct((NUM_SEG, DIM), jnp.float32),
)


@jax.jit
def kernel(x, batch):
    sums, cnts = _segment_sums(x, batch.astype(jnp.int32))
    return _combine(sums, cnts)


# speedup vs baseline: 2.5452x; 2.5452x over previous
import functools

import jax
import jax.numpy as jnp
from jax import lax
from jax.experimental import pallas as pl
from jax.experimental.pallas import tpu as pltpu
from jax.experimental.pallas import tpu_sc as plsc

NUM_SEG = 128
DIM = 512
LANES = 16
VECS = DIM // LANES
NUM_ROWS = 100000
CHUNK = 32
NUM_CHUNKS = NUM_ROWS // CHUNK
NUM_CORES = 2
NUM_SUBCORES = 16
NUM_WORKERS = NUM_CORES * NUM_SUBCORES
ITERS = -(-NUM_CHUNKS // NUM_WORKERS)

_mesh = plsc.VectorSubcoreMesh(core_axis_name="c", subcore_axis_name="s")


@functools.partial(
    pl.kernel,
    mesh=_mesh,
    out_type=(
        jax.ShapeDtypeStruct((NUM_WORKERS, NUM_SEG, DIM), jnp.float32),
        jax.ShapeDtypeStruct((NUM_WORKERS, NUM_SEG, LANES), jnp.float32),
    ),
    scratch_types=[
        pltpu.VMEM((CHUNK, DIM), jnp.float32),
        pltpu.VMEM((CHUNK, DIM), jnp.float32),
        pltpu.VMEM((CHUNK,), jnp.int32),
        pltpu.VMEM((CHUNK,), jnp.int32),
        pltpu.VMEM((NUM_SEG, DIM), jnp.float32),
        pltpu.VMEM((NUM_SEG, LANES), jnp.float32),
        pltpu.SemaphoreType.DMA,
        pltpu.SemaphoreType.DMA,
        pltpu.SemaphoreType.DMA,
        pltpu.SemaphoreType.DMA,
    ],
)
def _segment_sums(x_hbm, b_hbm, sums_hbm, cnts_hbm,
                  buf0, buf1, bidx0, bidx1, acc, cnt,
                  sx0, sx1, sb0, sb1):
    cid = lax.axis_index("c")
    sid = lax.axis_index("s")
    wid = sid * NUM_CORES + cid

    ones16 = jnp.full((LANES,), 1.0, jnp.float32)
    zero16 = jnp.zeros((LANES,), jnp.float32)

    def _zero_seg(g, carry):
        for j in range(VECS):
            acc[g, pl.ds(j * LANES, LANES)] = zero16
        cnt[g] = zero16
        return carry

    lax.fori_loop(0, NUM_SEG, _zero_seg, 0)

    def _start(k, buf, bidx, sx, sb):
        @pl.when(k < NUM_CHUNKS)
        def _():
            r0 = k * CHUNK
            pltpu.async_copy(b_hbm.at[pl.ds(r0, CHUNK)], bidx, sb)
            pltpu.async_copy(x_hbm.at[pl.ds(r0, CHUNK), :], buf, sx)

    def _finish(k, buf, bidx, sx, sb):
        @pl.when(k < NUM_CHUNKS)
        def _():
            pltpu.make_async_copy(b_hbm.at[pl.ds(k * CHUNK, CHUNK)],
                                  bidx, sb).wait()
            pltpu.make_async_copy(x_hbm.at[pl.ds(k * CHUNK, CHUNK), :],
                                  buf, sx).wait()

            bvec0 = bidx[pl.ds(0, LANES)]
            bvec1 = bidx[pl.ds(LANES, LANES)]
            b_first = bvec0[0]
            b_last = bvec1[LANES - 1]

            @pl.when(b_first == b_last)
            def _():
                def _col(j, c2):
                    sub = []
                    for rb in range(0, CHUNK, 8):
                        t01 = (buf[rb + 0, pl.ds(j * LANES, LANES)]
                               + buf[rb + 1, pl.ds(j * LANES, LANES)])
                        t23 = (buf[rb + 2, pl.ds(j * LANES, LANES)]
                               + buf[rb + 3, pl.ds(j * LANES, LANES)])
                        t45 = (buf[rb + 4, pl.ds(j * LANES, LANES)]
                               + buf[rb + 5, pl.ds(j * LANES, LANES)])
                        t67 = (buf[rb + 6, pl.ds(j * LANES, LANES)]
                               + buf[rb + 7, pl.ds(j * LANES, LANES)])
                        sub.append((t01 + t23) + (t45 + t67))
                    total = (sub[0] + sub[1]) + (sub[2] + sub[3])
                    plsc.addupdate(acc.at[b_first, pl.ds(j * LANES, LANES)],
                                   total)
                    return c2

                lax.fori_loop(0, VECS, _col, 0)
                plsc.addupdate(cnt.at[b_first], ones16 * float(CHUNK))

            @pl.when(b_first != b_last)
            def _():
                def _group(g, c2):
                    bvec = bidx[pl.ds(g * LANES, LANES)]
                    for r in range(LANES):
                        b = bvec[r]
                        row = g * LANES + r
                        for j in range(VECS):
                            v = buf[row, pl.ds(j * LANES, LANES)]
                            plsc.addupdate(
                                acc.at[b, pl.ds(j * LANES, LANES)], v)
                        plsc.addupdate(cnt.at[b], ones16)
                    return c2

                lax.fori_loop(0, CHUNK // LANES, _group, 0)

    _start(wid, buf0, bidx0, sx0, sb0)

    def _outer(t, carry):
        k0 = wid + NUM_WORKERS * (2 * t)
        k1 = wid + NUM_WORKERS * (2 * t + 1)
        _start(k1, buf1, bidx1, sx1, sb1)
        _finish(k0, buf0, bidx0, sx0, sb0)
        _start(k0 + 2 * NUM_WORKERS, buf0, bidx0, sx0, sb0)
        _finish(k1, buf1, bidx1, sx1, sb1)
        return carry

    lax.fori_loop(0, ITERS // 2, _outer, 0)

    pltpu.sync_copy(acc, sums_hbm.at[wid])
    pltpu.sync_copy(cnt, cnts_hbm.at[wid])


def _combine_body(s_ref, c_ref, o_ref):
    s = jnp.sum(s_ref[...], axis=0)
    c = jnp.sum(c_ref[...], axis=0)[:, 0:1]
    o_ref[...] = s / jnp.maximum(c, 1.0)


_combine = pl.pallas_call(
    _combine_body,
    out_shape=jax.ShapeDtypeStruct((NUM_SEG, DIM), jnp.float32),
)


@jax.jit
def kernel(x, batch):
    sums, cnts = _segment_sums(x, batch.astype(jnp.int32))
    return _combine(sums, cnts)


# speedup vs baseline: 2.6055x; 1.0237x over previous
import functools

import jax
import jax.numpy as jnp
from jax import lax
from jax.experimental import pallas as pl
from jax.experimental.pallas import tpu as pltpu
from jax.experimental.pallas import tpu_sc as plsc

NUM_SEG = 128
DIM = 512
LANES = 16
VECS = DIM // LANES
NUM_ROWS = 100000
CHUNK = 32
NUM_CHUNKS = NUM_ROWS // CHUNK
NUM_CORES = 2
NUM_SUBCORES = 16
NUM_WORKERS = NUM_CORES * NUM_SUBCORES
MAX_TILE_CHUNKS = -(-NUM_CHUNKS // NUM_WORKERS)
EXTRA = NUM_CHUNKS - (MAX_TILE_CHUNKS - 1) * NUM_WORKERS
BIDX_ROWS = MAX_TILE_CHUNKS * CHUNK

_mesh = plsc.VectorSubcoreMesh(core_axis_name="c", subcore_axis_name="s")


@functools.partial(
    pl.kernel,
    mesh=_mesh,
    out_type=(
        jax.ShapeDtypeStruct((NUM_WORKERS, NUM_SEG, DIM), jnp.float32),
        jax.ShapeDtypeStruct((NUM_WORKERS, NUM_SEG, LANES), jnp.float32),
    ),
    scratch_types=[
        pltpu.VMEM((CHUNK, DIM), jnp.float32),
        pltpu.VMEM((CHUNK, DIM), jnp.float32),
        pltpu.VMEM((BIDX_ROWS,), jnp.int32),
        pltpu.VMEM((NUM_SEG, DIM), jnp.float32),
        pltpu.VMEM((NUM_SEG, LANES), jnp.float32),
        pltpu.SemaphoreType.DMA,
        pltpu.SemaphoreType.DMA,
    ],
)
def _segment_sums(x_hbm, b_hbm, sums_hbm, cnts_hbm,
                  buf0, buf1, bidx, acc, cnt, sx0, sx1):
    cid = lax.axis_index("c")
    sid = lax.axis_index("s")
    wid = sid * NUM_CORES + cid

    c0 = (MAX_TILE_CHUNKS - 1) * wid + jnp.minimum(wid, EXTRA)
    n_chunks = jnp.where(wid < EXTRA, MAX_TILE_CHUNKS, MAX_TILE_CHUNKS - 1)
    sc = jnp.minimum(c0, NUM_CHUNKS - MAX_TILE_CHUNKS)

    ones16 = jnp.full((LANES,), 1.0, jnp.float32)
    zero16 = jnp.zeros((LANES,), jnp.float32)

    pltpu.sync_copy(b_hbm.at[pl.ds(sc * CHUNK, BIDX_ROWS)], bidx)

    def _zero_seg(g, carry):
        for j in range(VECS):
            acc[g, pl.ds(j * LANES, LANES)] = zero16
        cnt[g] = zero16
        return carry

    lax.fori_loop(0, NUM_SEG, _zero_seg, 0)

    def _start(i, buf, sx):
        @pl.when(i < n_chunks)
        def _():
            r0 = (c0 + i) * CHUNK
            pltpu.async_copy(x_hbm.at[pl.ds(r0, CHUNK), :], buf, sx)

    def _finish(i, buf, sx):
        @pl.when(i < n_chunks)
        def _():
            r0 = (c0 + i) * CHUNK
            lr = (c0 + i - sc) * CHUNK
            pltpu.make_async_copy(x_hbm.at[pl.ds(r0, CHUNK), :],
                                  buf, sx).wait()

            bvec0 = bidx[pl.ds(lr, LANES)]
            bvec1 = bidx[pl.ds(lr + LANES, LANES)]
            b_first = bvec0[0]
            b_last = bvec1[LANES - 1]

            @pl.when(b_first == b_last)
            def _():
                def _col(j, c2):
                    sub = []
                    for rb in range(0, CHUNK, 8):
                        t01 = (buf[rb + 0, pl.ds(j * LANES, LANES)]
                               + buf[rb + 1, pl.ds(j * LANES, LANES)])
                        t23 = (buf[rb + 2, pl.ds(j * LANES, LANES)]
                               + buf[rb + 3, pl.ds(j * LANES, LANES)])
                        t45 = (buf[rb + 4, pl.ds(j * LANES, LANES)]
                               + buf[rb + 5, pl.ds(j * LANES, LANES)])
                        t67 = (buf[rb + 6, pl.ds(j * LANES, LANES)]
                               + buf[rb + 7, pl.ds(j * LANES, LANES)])
                        sub.append((t01 + t23) + (t45 + t67))
                    total = (sub[0] + sub[1]) + (sub[2] + sub[3])
                    plsc.addupdate(acc.at[b_first, pl.ds(j * LANES, LANES)],
                                   total)
                    return c2

                lax.fori_loop(0, VECS, _col, 0, unroll=2)
                plsc.addupdate(cnt.at[b_first], ones16 * float(CHUNK))

            @pl.when(b_first != b_last)
            def _():
                def _group(g, c2):
                    bvec = bidx[pl.ds(lr + g * LANES, LANES)]
                    for r in range(LANES):
                        b = bvec[r]
                        row = g * LANES + r
                        for j in range(VECS):
                            v = buf[row, pl.ds(j * LANES, LANES)]
                            plsc.addupdate(
                                acc.at[b, pl.ds(j * LANES, LANES)], v)
                        plsc.addupdate(cnt.at[b], ones16)
                    return c2

                lax.fori_loop(0, CHUNK // LANES, _group, 0)

    _start(0, buf0, sx0)

    def _outer(t, carry):
        i0 = 2 * t
        i1 = 2 * t + 1
        _start(i1, buf1, sx1)
        _finish(i0, buf0, sx0)
        _start(i0 + 2, buf0, sx0)
        _finish(i1, buf1, sx1)
        return carry

    lax.fori_loop(0, MAX_TILE_CHUNKS // 2, _outer, 0)

    pltpu.sync_copy(acc, sums_hbm.at[wid])
    pltpu.sync_copy(cnt, cnts_hbm.at[wid])


def _combine_body(s_ref, c_ref, o_ref):
    s = jnp.sum(s_ref[...], axis=0)
    c = jnp.sum(c_ref[...], axis=0)[:, 0:1]
    o_ref[...] = s / jnp.maximum(c, 1.0)


_combine = pl.pallas_call(
    _combine_body,
    out_shape=jax.ShapeDtypeStruct((NUM_SEG, DIM), jnp.float32),
)


@jax.jit
def kernel(x, batch):
    sums, cnts = _segment_sums(x, batch.astype(jnp.int32))
    return _combine(sums, cnts)
